# baseline (speedup 1.0000x reference)
import functools

import jax
import jax.numpy as jnp
from jax import lax
from jax.experimental import pallas as pl
from jax.experimental.pallas import tpu as pltpu
from jax.experimental.pallas import tpu_sc as plsc

B = 16384
NC, NS, L = 2, 16, 16
NW = NC * NS
BW = B // NW
NCHUNK = BW // L

OFFS = (0, 7, 13, 25, 29, 35, 40, 44)
TBL = 58


def _body(cs, cu, cc, br, ga, gs, gz, gc, w_hbm, b_hbm, out_hbm,
          idx_v, w_v, b_v, d_v, out_v, sem):
    wid = lax.axis_index("s") * NC + lax.axis_index("c")
    base = wid * BW

    feats = (cs, cu, cc, br, ga, gs, gz, gc)
    copies = [pltpu.async_copy(feats[f].at[pl.ds(base, BW)], idx_v.at[f], sem)
              for f in range(8)]
    copies.append(pltpu.async_copy(w_hbm, w_v, sem))
    copies.append(pltpu.async_copy(b_hbm, b_v, sem))
    for c in copies:
        c.wait()

    lane = lax.iota(jnp.int32, L)
    zero = jnp.zeros((L,), jnp.int32)

    for j in range(4):
        r = jnp.minimum(lane + (L * j), TBL - 1)
        w0 = plsc.load_gather(w_v, [r * 2])
        w1 = plsc.load_gather(w_v, [r * 2 + 1])
        d_v[pl.ds(L * j, L)] = w0 - w1

    bd = plsc.load_gather(b_v, [zero]) - plsc.load_gather(b_v, [zero + 1])

    for c in range(NCHUNK):
        acc = bd
        for f in range(8):
            x = idx_v[f, pl.ds(c * L, L)]
            acc = acc + plsc.load_gather(d_v, [x + OFFS[f]])
        p0 = 1.0 / (1.0 + jnp.exp(-acc))
        p1 = 1.0 - p0
        ids = lane * 2 + (c * 2 * L)
        plsc.store_scatter(out_v, [ids], p0)
        plsc.store_scatter(out_v, [ids + 1], p1)

    pltpu.sync_copy(out_v, out_hbm.at[pl.ds(base * 2, BW * 2)])


_mushroom_sc = functools.partial(
    pl.kernel,
    out_type=jax.ShapeDtypeStruct((B * 2,), jnp.float32),
    mesh=plsc.VectorSubcoreMesh(core_axis_name="c", subcore_axis_name="s"),
    compiler_params=pltpu.CompilerParams(needs_layout_passes=False),
    scratch_types=[
        pltpu.VMEM((8, BW), jnp.int32),
        pltpu.VMEM((128,), jnp.float32),
        pltpu.VMEM((16,), jnp.float32),
        pltpu.VMEM((64,), jnp.float32),
        pltpu.VMEM((BW * 2,), jnp.float32),
        pltpu.SemaphoreType.DMA,
    ],
)(_body)


def kernel(cap_shape, cap_surface, cap_color, bruises, odor, gill_attachment,
           gill_spacing, gill_size, gill_color, W, b):
    del odor
    idxs = [x.astype(jnp.int32) for x in
            (cap_shape, cap_surface, cap_color, bruises, gill_attachment,
             gill_spacing, gill_size, gill_color)]
    w_flat = jnp.pad(W.astype(jnp.float32).reshape(-1), (0, 128 - 2 * TBL))
    b_pad = jnp.pad(b.astype(jnp.float32), (0, 14))
    out = _mushroom_sc(*idxs, w_flat, b_pad)
    return out.reshape(B, 2)


# speedup vs baseline: 1.0018x; 1.0018x over previous
import functools

import jax
import jax.numpy as jnp
from jax import lax
from jax.experimental import pallas as pl
from jax.experimental.pallas import tpu as pltpu
from jax.experimental.pallas import tpu_sc as plsc

B = 16384
NC, NS, L = 2, 16, 16
NW = NC * NS
BW = B // NW
NCHUNK = BW // L

OFFS = (0, 7, 13, 25, 29, 35, 40, 44)
TBL = 58


def _body(cs, cu, cc, br, ga, gs, gz, gc, w_hbm, b_hbm, out_hbm,
          idx_v, w_v, b_v, d_v, out_v, sem):
    wid = lax.axis_index("s") * NC + lax.axis_index("c")
    base = wid * BW

    feats = (cs, cu, cc, br, ga, gs, gz, gc)
    copies = [pltpu.async_copy(feats[f].at[pl.ds(base, BW)], idx_v.at[f], sem)
              for f in range(8)]
    copies.append(pltpu.async_copy(w_hbm, w_v, sem))
    copies.append(pltpu.async_copy(b_hbm, b_v, sem))
    for c in copies:
        c.wait()

    lane = lax.iota(jnp.int32, L)
    zero = jnp.zeros((L,), jnp.int32)

    for j in range(4):
        r = jnp.minimum(lane + (L * j), TBL - 1)
        w0 = plsc.load_gather(w_v, [r * 2])
        w1 = plsc.load_gather(w_v, [r * 2 + 1])
        d_v[pl.ds(L * j, L)] = w0 - w1

    bd = plsc.load_gather(b_v, [zero]) - plsc.load_gather(b_v, [zero + 1])

    for c in range(NCHUNK):
        acc = bd
        for f in range(8):
            x = idx_v[f, pl.ds(c * L, L)]
            acc = acc + plsc.load_gather(d_v, [x + OFFS[f]])
        p0 = 1.0 / (1.0 + jnp.exp(-acc))
        p1 = 1.0 - p0
        ids = lane * 2 + (c * 2 * L)
        plsc.store_scatter(out_v, [ids], p0)
        plsc.store_scatter(out_v, [ids + 1], p1)

    pltpu.sync_copy(out_v, out_hbm.at[pl.ds(base * 2, BW * 2)])


_mushroom_sc = functools.partial(
    pl.kernel,
    out_type=jax.ShapeDtypeStruct((B * 2,), jnp.float32),
    mesh=plsc.VectorSubcoreMesh(core_axis_name="c", subcore_axis_name="s"),
    compiler_params=pltpu.CompilerParams(
        needs_layout_passes=False,
        disable_bounds_checks=True,
        disable_semaphore_checks=True,
        skip_device_barrier=True,
    ),
    scratch_types=[
        pltpu.VMEM((8, BW), jnp.int32),
        pltpu.VMEM((128,), jnp.float32),
        pltpu.VMEM((16,), jnp.float32),
        pltpu.VMEM((64,), jnp.float32),
        pltpu.VMEM((BW * 2,), jnp.float32),
        pltpu.SemaphoreType.DMA,
    ],
)(_body)


def kernel(cap_shape, cap_surface, cap_color, bruises, odor, gill_attachment,
           gill_spacing, gill_size, gill_color, W, b):
    del odor
    idxs = [x.astype(jnp.int32) for x in
            (cap_shape, cap_surface, cap_color, bruises, gill_attachment,
             gill_spacing, gill_size, gill_color)]
    w_flat = jnp.pad(W.astype(jnp.float32).reshape(-1), (0, 128 - 2 * TBL))
    b_pad = jnp.pad(b.astype(jnp.float32), (0, 14))
    out = _mushroom_sc(*idxs, w_flat, b_pad)
    return out.reshape(B, 2)


# speedup vs baseline: 1.1961x; 1.1940x over previous
import functools

import jax
import jax.numpy as jnp
from jax import lax
from jax.experimental import pallas as pl
from jax.experimental.pallas import tpu as pltpu
from jax.experimental.pallas import tpu_sc as plsc

B = 16384
NC, NS, L = 2, 16, 16
NW = NC * NS
BW = B // NW
NCHUNK = BW // L

OFFS = (0, 7, 13, 25, 29, 35, 40, 44)
TBL = 58


def _body(cs, cu, cc, br, ga, gs, gz, gc, w_hbm, b_hbm, out_hbm,
          idx_v, w_v, b_v, d_v, out_v, sem):
    wid = lax.axis_index("s") * NC + lax.axis_index("c")
    base = wid * BW

    feats = (cs, cu, cc, br, ga, gs, gz, gc)
    copies = [pltpu.async_copy(feats[f].at[pl.ds(base, BW)], idx_v.at[f], sem)
              for f in range(8)]
    copies.append(pltpu.async_copy(w_hbm, w_v, sem))
    copies.append(pltpu.async_copy(b_hbm, b_v, sem))
    for c in copies:
        c.wait()

    lane = lax.iota(jnp.int32, L)
    zero = jnp.zeros((L,), jnp.int32)

    for j in range(4):
        r = jnp.minimum(lane + (L * j), TBL - 1)
        w0 = plsc.load_gather(w_v, [r, zero])
        w1 = plsc.load_gather(w_v, [r, zero + 1])
        d_v[pl.ds(L * j, L)] = w0 - w1

    bd = plsc.load_gather(b_v, [zero]) - plsc.load_gather(b_v, [zero + 1])

    for c in range(NCHUNK):
        acc = bd
        for f in range(8):
            x = idx_v[f, pl.ds(c * L, L)]
            acc = acc + plsc.load_gather(d_v, [x + OFFS[f]])
        p0 = 1.0 / (1.0 + jnp.exp(-acc))
        p1 = 1.0 - p0
        row = lane + (c * L)
        plsc.store_scatter(out_v, [row, zero], p0)
        plsc.store_scatter(out_v, [row, zero + 1], p1)

    pltpu.sync_copy(out_v, out_hbm.at[pl.ds(base, BW), :])


_mushroom_sc = functools.partial(
    pl.kernel,
    out_type=jax.ShapeDtypeStruct((B, 2), jnp.float32),
    mesh=plsc.VectorSubcoreMesh(core_axis_name="c", subcore_axis_name="s"),
    compiler_params=pltpu.CompilerParams(
        needs_layout_passes=False,
        disable_bounds_checks=True,
        disable_semaphore_checks=True,
        skip_device_barrier=True,
    ),
    scratch_types=[
        pltpu.VMEM((8, BW), jnp.int32),
        pltpu.VMEM((TBL, 2), jnp.float32),
        pltpu.VMEM((2,), jnp.float32),
        pltpu.VMEM((64,), jnp.float32),
        pltpu.VMEM((BW, 2), jnp.float32),
        pltpu.SemaphoreType.DMA,
    ],
)(_body)


def kernel(cap_shape, cap_surface, cap_color, bruises, odor, gill_attachment,
           gill_spacing, gill_size, gill_color, W, b):
    del odor
    idxs = [x.astype(jnp.int32) for x in
            (cap_shape, cap_surface, cap_color, bruises, gill_attachment,
             gill_spacing, gill_size, gill_color)]
    return _mushroom_sc(*idxs, W.astype(jnp.float32), b.astype(jnp.float32))


# speedup vs baseline: 1.5271x; 1.2767x over previous
import functools

import jax
import jax.numpy as jnp
from jax import lax
from jax.experimental import pallas as pl
from jax.experimental.pallas import tpu as pltpu
from jax.experimental.pallas import tpu_sc as plsc

B = 16384
NC, NS, L = 2, 16, 16
NW = NC * NS
BW = B // NW
NCHUNK = BW // L

OFFS = (0, 7, 13, 25, 29, 35, 40, 44)
TBL = 58


def _body(cs, cu, cc, br, ga, gs, gz, gc, w_hbm, b_hbm, p0_hbm, p1_hbm,
          idx_v, w_v, b_v, d_v, p0_v, p1_v, sem):
    wid = lax.axis_index("s") * NC + lax.axis_index("c")
    base = wid * BW

    feats = (cs, cu, cc, br, ga, gs, gz, gc)
    copies = [pltpu.async_copy(feats[f].at[pl.ds(base, BW)], idx_v.at[f], sem)
              for f in range(8)]
    copies.append(pltpu.async_copy(w_hbm, w_v, sem))
    copies.append(pltpu.async_copy(b_hbm, b_v, sem))
    for c in copies:
        c.wait()

    lane = lax.iota(jnp.int32, L)
    zero = jnp.zeros((L,), jnp.int32)

    for j in range(4):
        r = jnp.minimum(lane + (L * j), TBL - 1)
        w0 = plsc.load_gather(w_v, [r, zero])
        w1 = plsc.load_gather(w_v, [r, zero + 1])
        d_v[pl.ds(L * j, L)] = w0 - w1

    bd = plsc.load_gather(b_v, [zero]) - plsc.load_gather(b_v, [zero + 1])

    for c in range(NCHUNK):
        acc = bd
        for f in range(8):
            x = idx_v[f, pl.ds(c * L, L)]
            acc = acc + plsc.load_gather(d_v, [x + OFFS[f]])
        p0 = 1.0 / (1.0 + jnp.exp(-acc))
        p0_v[pl.ds(c * L, L)] = p0
        p1_v[pl.ds(c * L, L)] = 1.0 - p0

    out0 = pltpu.async_copy(p0_v, p0_hbm.at[pl.ds(base, BW)], sem)
    out1 = pltpu.async_copy(p1_v, p1_hbm.at[pl.ds(base, BW)], sem)
    out0.wait()
    out1.wait()


_mushroom_sc = functools.partial(
    pl.kernel,
    out_type=(jax.ShapeDtypeStruct((B,), jnp.float32),
              jax.ShapeDtypeStruct((B,), jnp.float32)),
    mesh=plsc.VectorSubcoreMesh(core_axis_name="c", subcore_axis_name="s"),
    compiler_params=pltpu.CompilerParams(
        needs_layout_passes=False,
        disable_bounds_checks=True,
        disable_semaphore_checks=True,
        skip_device_barrier=True,
    ),
    scratch_types=[
        pltpu.VMEM((8, BW), jnp.int32),
        pltpu.VMEM((TBL, 2), jnp.float32),
        pltpu.VMEM((2,), jnp.float32),
        pltpu.VMEM((64,), jnp.float32),
        pltpu.VMEM((BW,), jnp.float32),
        pltpu.VMEM((BW,), jnp.float32),
        pltpu.SemaphoreType.DMA,
    ],
)(_body)


def kernel(cap_shape, cap_surface, cap_color, bruises, odor, gill_attachment,
           gill_spacing, gill_size, gill_color, W, b):
    del odor
    idxs = [x.astype(jnp.int32) for x in
            (cap_shape, cap_surface, cap_color, bruises, gill_attachment,
             gill_spacing, gill_size, gill_color)]
    p0, p1 = _mushroom_sc(*idxs, W.astype(jnp.float32), b.astype(jnp.float32))
    return jnp.stack([p0, p1], axis=1)


# speedup vs baseline: 1.6998x; 1.1130x over previous
import functools

import jax
import jax.numpy as jnp
from jax import lax
from jax.experimental import pallas as pl
from jax.experimental.pallas import tpu as pltpu
from jax.experimental.pallas import tpu_sc as plsc

B = 16384
NC, NS, L = 2, 16, 16
NW = NC * NS
BW = B // NW
NCHUNK = BW // L

OFFS = (0, 7, 13, 25, 29, 35, 40, 44)
TBL = 58


def _body(cs, cu, cc, br, ga, gs, gz, gc, w_hbm, b_hbm, p0_hbm, p1_hbm,
          idx_v, w_v, b_v, d_v, p0_v, p1_v, sem):
    wid = lax.axis_index("s") * NC + lax.axis_index("c")
    base = wid * BW

    feats = (cs, cu, cc, br, ga, gs, gz, gc)
    copies = [pltpu.async_copy(feats[f].at[pl.ds(base, BW)], idx_v.at[f], sem)
              for f in range(8)]
    copies.append(pltpu.async_copy(w_hbm, w_v, sem))
    copies.append(pltpu.async_copy(b_hbm, b_v, sem))
    for c in copies:
        c.wait()

    lane = lax.iota(jnp.int32, L)
    zero = jnp.zeros((L,), jnp.int32)

    for j in range(4):
        r = jnp.minimum(lane + (L * j), TBL - 1)
        w0 = plsc.load_gather(w_v, [r, zero])
        w1 = plsc.load_gather(w_v, [r, zero + 1])
        d_v[pl.ds(L * j, L)] = w0 - w1

    bd = plsc.load_gather(b_v, [zero]) - plsc.load_gather(b_v, [zero + 1])

    @plsc.parallel_loop(0, BW, step=L)
    def _chunk(i):
        acc = bd
        for f in range(8):
            x = idx_v[f, pl.ds(i, L)]
            acc = acc + plsc.load_gather(d_v, [x + OFFS[f]])
        p0 = 1.0 / (1.0 + jnp.exp(-acc))
        p0_v[pl.ds(i, L)] = p0
        p1_v[pl.ds(i, L)] = 1.0 - p0

    out0 = pltpu.async_copy(p0_v, p0_hbm.at[pl.ds(base, BW)], sem)
    out1 = pltpu.async_copy(p1_v, p1_hbm.at[pl.ds(base, BW)], sem)
    out0.wait()
    out1.wait()


_mushroom_sc = functools.partial(
    pl.kernel,
    out_type=(jax.ShapeDtypeStruct((B,), jnp.float32),
              jax.ShapeDtypeStruct((B,), jnp.float32)),
    mesh=plsc.VectorSubcoreMesh(core_axis_name="c", subcore_axis_name="s"),
    compiler_params=pltpu.CompilerParams(
        needs_layout_passes=False,
        disable_bounds_checks=True,
        disable_semaphore_checks=True,
        skip_device_barrier=True,
    ),
    scratch_types=[
        pltpu.VMEM((8, BW), jnp.int32),
        pltpu.VMEM((TBL, 2), jnp.float32),
        pltpu.VMEM((2,), jnp.float32),
        pltpu.VMEM((64,), jnp.float32),
        pltpu.VMEM((BW,), jnp.float32),
        pltpu.VMEM((BW,), jnp.float32),
        pltpu.SemaphoreType.DMA,
    ],
)(_body)


def kernel(cap_shape, cap_surface, cap_color, bruises, odor, gill_attachment,
           gill_spacing, gill_size, gill_color, W, b):
    del odor
    idxs = [x.astype(jnp.int32) for x in
            (cap_shape, cap_surface, cap_color, bruises, gill_attachment,
             gill_spacing, gill_size, gill_color)]
    p0, p1 = _mushroom_sc(*idxs, W.astype(jnp.float32), b.astype(jnp.float32))
    return jnp.stack([p0, p1], axis=1)


# speedup vs baseline: 1.8059x; 1.0624x over previous
import functools

import jax
import jax.numpy as jnp
from jax import lax
from jax.experimental import pallas as pl
from jax.experimental.pallas import tpu as pltpu
from jax.experimental.pallas import tpu_sc as plsc

B = 16384
NC, NS, L = 2, 16, 16
NW = 1 * NS
BW = B // NW
NCHUNK = BW // L

OFFS = (0, 7, 13, 25, 29, 35, 40, 44)
TBL = 58


def _body(cs, cu, cc, br, ga, gs, gz, gc, w_hbm, b_hbm, p0_hbm, p1_hbm,
          idx_v, w_v, b_v, d_v, p0_v, p1_v, sem):
    wid = lax.axis_index("s")
    base = wid * BW

    feats = (cs, cu, cc, br, ga, gs, gz, gc)
    copies = [pltpu.async_copy(feats[f].at[pl.ds(base, BW)], idx_v.at[f], sem)
              for f in range(8)]
    copies.append(pltpu.async_copy(w_hbm, w_v, sem))
    copies.append(pltpu.async_copy(b_hbm, b_v, sem))
    for c in copies:
        c.wait()

    lane = lax.iota(jnp.int32, L)
    zero = jnp.zeros((L,), jnp.int32)

    for j in range(4):
        r = jnp.minimum(lane + (L * j), TBL - 1)
        w0 = plsc.load_gather(w_v, [r, zero])
        w1 = plsc.load_gather(w_v, [r, zero + 1])
        d_v[pl.ds(L * j, L)] = w0 - w1

    bd = plsc.load_gather(b_v, [zero]) - plsc.load_gather(b_v, [zero + 1])

    @plsc.parallel_loop(0, BW, step=L)
    def _chunk(i):
        acc = bd
        for f in range(8):
            x = idx_v[f, pl.ds(i, L)]
            acc = acc + plsc.load_gather(d_v, [x + OFFS[f]])
        p0 = 1.0 / (1.0 + jnp.exp(-acc))
        p0_v[pl.ds(i, L)] = p0
        p1_v[pl.ds(i, L)] = 1.0 - p0

    out0 = pltpu.async_copy(p0_v, p0_hbm.at[pl.ds(base, BW)], sem)
    out1 = pltpu.async_copy(p1_v, p1_hbm.at[pl.ds(base, BW)], sem)
    out0.wait()
    out1.wait()


_mushroom_sc = functools.partial(
    pl.kernel,
    out_type=(jax.ShapeDtypeStruct((B,), jnp.float32),
              jax.ShapeDtypeStruct((B,), jnp.float32)),
    mesh=plsc.VectorSubcoreMesh(core_axis_name="c", subcore_axis_name="s", num_cores=1),
    compiler_params=pltpu.CompilerParams(
        needs_layout_passes=False,
        disable_bounds_checks=True,
        disable_semaphore_checks=True,
        skip_device_barrier=True,
    ),
    scratch_types=[
        pltpu.VMEM((8, BW), jnp.int32),
        pltpu.VMEM((TBL, 2), jnp.float32),
        pltpu.VMEM((2,), jnp.float32),
        pltpu.VMEM((64,), jnp.float32),
        pltpu.VMEM((BW,), jnp.float32),
        pltpu.VMEM((BW,), jnp.float32),
        pltpu.SemaphoreType.DMA,
    ],
)(_body)


def kernel(cap_shape, cap_surface, cap_color, bruises, odor, gill_attachment,
           gill_spacing, gill_size, gill_color, W, b):
    del odor
    idxs = [x.astype(jnp.int32) for x in
            (cap_shape, cap_surface, cap_color, bruises, gill_attachment,
             gill_spacing, gill_size, gill_color)]
    p0, p1 = _mushroom_sc(*idxs, W.astype(jnp.float32), b.astype(jnp.float32))
    return jnp.stack([p0, p1], axis=1)
